# baseline (speedup 1.0000x reference)
import jax
import jax.numpy as jnp
from jax import lax
from jax.experimental import pallas as pl
from jax.experimental.pallas import tpu as pltpu
from jax.experimental.pallas import tpu_sc as plsc

N = 320000
D = 128
S = 10000
S_PAD = 10240
NC = 2
NS = 16
NW = NC * NS
ROWS_PER_TILE = N // NW
SCAT = 64
NLOAD = ROWS_PER_TILE // SCAT
REM = ROWS_PER_TILE - NLOAD * SCAT
NBUF = 5
DL = 3
DS = 2
SEG_PER_TILE = S_PAD // NS


def _sc_body(x_hbm, b_hbm, sums_out, cnts_out, acc, cnt, *rest):
    xb = rest[0:NBUF]
    ix = rest[NBUF:2 * NBUF]
    ones, zc, ixr, onesr = rest[2 * NBUF:2 * NBUF + 4]
    semL = rest[2 * NBUF + 4:3 * NBUF + 4]
    semS = rest[3 * NBUF + 4:4 * NBUF + 4]
    semC = rest[4 * NBUF + 4:5 * NBUF + 4]
    xb0 = xb[0]

    cid = lax.axis_index("c")
    sid = lax.axis_index("s")
    wid = cid * NS + sid
    base = wid * ROWS_PER_TILE

    def _zx(i, _):
        xb0[i // 8, pl.ds((i % 8) * 16, 16)] = jnp.zeros((16,), jnp.float32)
        return 0
    lax.fori_loop(0, SCAT * 8, _zx, 0)

    def _zc(i, _):
        zc[pl.ds(i * 16, 16)] = jnp.zeros((16,), jnp.float32)
        return 0
    lax.fori_loop(0, SEG_PER_TILE // 16, _zc, 0)

    def _on(i, _):
        ones[pl.ds(i * 16, 16)] = jnp.ones((16,), jnp.float32)
        return 0
    lax.fori_loop(0, SCAT // 16, _on, 0)
    onesr[pl.ds(0, 16)] = jnp.ones((16,), jnp.float32)

    s0 = sid * SEG_PER_TILE
    for k in range(SEG_PER_TILE // SCAT):
        pltpu.sync_copy(xb0, acc.at[pl.ds(s0 + k * SCAT, SCAT)])
    pltpu.sync_copy(zc, cnt.at[pl.ds(s0, SEG_PER_TILE)])

    def L_start(j, b):
        r = base + j * SCAT
        pltpu.async_copy(x_hbm.at[pl.ds(r, SCAT)], xb[b], semL[b])
        pltpu.async_copy(b_hbm.at[pl.ds(r, SCAT)], ix[b], semL[b])

    def L_wait(j, b):
        r = base + j * SCAT
        pltpu.make_async_copy(x_hbm.at[pl.ds(r, SCAT)], xb[b], semL[b]).wait()
        pltpu.make_async_copy(b_hbm.at[pl.ds(r, SCAT)], ix[b], semL[b]).wait()

    def S_start(j, b):
        pltpu.async_copy(xb[b], acc.at[ix[b]], semS[b], add=True)
        pltpu.async_copy(ones, cnt.at[ix[b]], semC[b], add=True)

    def S_wait(j, b):
        pltpu.make_async_copy(xb[b], acc.at[ix[b]], semS[b]).wait()
        pltpu.make_async_copy(ones, cnt.at[ix[b]], semC[b]).wait()

    def _line(j):
        b = j % NBUF
        L_wait(j, b)
        S_start(j, b)
        if j - DS >= 0:
            S_wait(j - DS, (j - DS) % NBUF)
        if j + DL <= NLOAD - 1:
            L_start(j + DL, (j + DL) % NBUF)

    for j0 in range(DL):
        L_start(j0, j0 % NBUF)
    plsc.subcore_barrier()

    HEAD = NBUF
    TAIL0 = ((NLOAD - DL) // NBUF) * NBUF
    for j0 in range(HEAD):
        _line(j0)

    def _steady(i, _):
        for b in range(NBUF):
            j = NBUF * i + b
            L_wait(j, b)
            S_start(j, b)
            S_wait(j - DS, (b - DS) % NBUF)
            L_start(j + DL, (b + DL) % NBUF)
        return 0
    lax.fori_loop(HEAD // NBUF, TAIL0 // NBUF, _steady, 0)

    for j0 in range(TAIL0, NLOAD):
        _line(j0)
    for j0 in range(NLOAD - DS, NLOAD):
        S_wait(j0, j0 % NBUF)

    rr = base + NLOAD * SCAT
    pltpu.sync_copy(b_hbm.at[pl.ds(rr, REM)], ixr)
    pltpu.sync_copy(x_hbm.at[pl.ds(rr, REM)], xb0.at[pl.ds(0, REM)])
    pltpu.sync_copy(xb0.at[pl.ds(0, REM)], acc.at[ixr], add=True)
    pltpu.sync_copy(onesr, cnt.at[ixr], add=True)
    plsc.subcore_barrier()

    pltpu.sync_copy(acc.at[pl.ds(s0, SEG_PER_TILE)],
                    sums_out.at[cid, pl.ds(s0, SEG_PER_TILE)])
    pltpu.sync_copy(cnt.at[pl.ds(s0, SEG_PER_TILE)],
                    cnts_out.at[cid, pl.ds(s0, SEG_PER_TILE)])


@jax.jit
def _sc_aggregate(x, batch):
    mesh = plsc.VectorSubcoreMesh(core_axis_name="c", subcore_axis_name="s")
    f = pl.kernel(
        _sc_body,
        out_type=(jax.ShapeDtypeStruct((NC, S_PAD, D), jnp.float32),
                  jax.ShapeDtypeStruct((NC, S_PAD), jnp.float32)),
        mesh=mesh,
        scratch_types=[
            pltpu.VMEM_SHARED((S_PAD, D), jnp.float32),
            pltpu.VMEM_SHARED((S_PAD,), jnp.float32),
        ] + [pltpu.VMEM((SCAT, D), jnp.float32)] * NBUF
          + [pltpu.VMEM((SCAT,), jnp.int32)] * NBUF + [
            pltpu.VMEM((SCAT,), jnp.float32),
            pltpu.VMEM((SEG_PER_TILE,), jnp.float32),
            pltpu.VMEM((REM,), jnp.int32),
            pltpu.VMEM((REM,), jnp.float32),
        ] + [pltpu.SemaphoreType.DMA] * (3 * NBUF),
    )
    return f(x, batch)


BLK = 2000


def _fin_body(sums_ref, cnts_ref, y_ref, w_ref, b_ref, o_ref):
    s = sums_ref[0] + sums_ref[1]
    cnt = cnts_ref[:, 0:1] + cnts_ref[:, 1:2]
    ctx = jnp.dot(y_ref[...], w_ref[...].T,
                  preferred_element_type=jnp.float32) + b_ref[...]
    mean = s / jnp.maximum(cnt, 1.0)
    o_ref[...] = mean + jnp.where(cnt > 0.0, ctx, 0.0)


@jax.jit
def _finalize(sums, cnts_t, y2, W_c, b2):
    return pl.pallas_call(
        _fin_body,
        grid=(S // BLK,),
        in_specs=[
            pl.BlockSpec((NC, BLK, D), lambda i: (0, i, 0)),
            pl.BlockSpec((BLK, NC), lambda i: (i, 0)),
            pl.BlockSpec((1, D), lambda i: (0, 0)),
            pl.BlockSpec((D, D), lambda i: (0, 0)),
            pl.BlockSpec((1, D), lambda i: (0, 0)),
        ],
        out_specs=pl.BlockSpec((BLK, D), lambda i: (i, 0)),
        out_shape=jax.ShapeDtypeStruct((S, D), jnp.float32),
    )(sums, cnts_t, y2, W_c, b2)


def kernel(x, y, batch, W_c, b_c):
    batch32 = batch.astype(jnp.int32)
    sums, cnts = _sc_aggregate(x, batch32)
    return _finalize(sums, cnts.T, y.reshape(1, D), W_c, b_c.reshape(1, D))


# speedup vs baseline: 1.0334x; 1.0334x over previous
import jax
import jax.numpy as jnp
from jax import lax
from jax.experimental import pallas as pl
from jax.experimental.pallas import tpu as pltpu
from jax.experimental.pallas import tpu_sc as plsc

N = 320000
D = 128
S = 10000
S_PAD = 10240
NC = 2
NS = 16
NW = NC * NS
ROWS_PER_TILE = N // NW
SCAT = 80
NLOAD = ROWS_PER_TILE // SCAT
REM = ROWS_PER_TILE - NLOAD * SCAT
NBUF = 4
DL = 3
DS = 1
SEG_PER_TILE = S_PAD // NS


def _sc_body(x_hbm, b_hbm, sums_out, cnts_out, acc, cnt, *rest):
    xb = rest[0:NBUF]
    ix = rest[NBUF:2 * NBUF]
    ones, zc, ixr, onesr = rest[2 * NBUF:2 * NBUF + 4]
    semL = rest[2 * NBUF + 4:3 * NBUF + 4]
    semS = rest[3 * NBUF + 4:4 * NBUF + 4]
    semC = rest[4 * NBUF + 4:5 * NBUF + 4]
    xb0 = xb[0]

    cid = lax.axis_index("c")
    sid = lax.axis_index("s")
    wid = cid * NS + sid
    base = wid * ROWS_PER_TILE

    zsrc = xb[NBUF - 1]

    def _zx(i, _):
        zsrc[i // 8, pl.ds((i % 8) * 16, 16)] = jnp.zeros((16,), jnp.float32)
        return 0
    lax.fori_loop(0, SCAT * 8, _zx, 0)

    def _zc(i, _):
        zc[pl.ds(i * 16, 16)] = jnp.zeros((16,), jnp.float32)
        return 0
    lax.fori_loop(0, SEG_PER_TILE // 16, _zc, 0)

    def _on(i, _):
        ones[pl.ds(i * 16, 16)] = jnp.ones((16,), jnp.float32)
        return 0
    lax.fori_loop(0, SCAT // 16, _on, 0)
    onesr[pl.ds(0, 16)] = jnp.ones((16,), jnp.float32)

    s0 = sid * SEG_PER_TILE
    NZ = SEG_PER_TILE // SCAT
    for k in range(NZ):
        pltpu.async_copy(zsrc, acc.at[pl.ds(s0 + k * SCAT, SCAT)],
                         semS[k % NBUF])
    pltpu.async_copy(zc, cnt.at[pl.ds(s0, SEG_PER_TILE)], semC[0])

    def L_start(j, b):
        r = base + j * SCAT
        pltpu.async_copy(x_hbm.at[pl.ds(r, SCAT)], xb[b], semL[b])
        pltpu.async_copy(b_hbm.at[pl.ds(r, SCAT)], ix[b], semL[b])

    def L_wait(j, b):
        r = base + j * SCAT
        pltpu.make_async_copy(x_hbm.at[pl.ds(r, SCAT)], xb[b], semL[b]).wait()
        pltpu.make_async_copy(b_hbm.at[pl.ds(r, SCAT)], ix[b], semL[b]).wait()

    def S_start(j, b):
        pltpu.async_copy(xb[b], acc.at[ix[b]], semS[b], add=True)
        pltpu.async_copy(ones, cnt.at[ix[b]], semC[b], add=True)

    def S_wait(j, b):
        pltpu.make_async_copy(xb[b], acc.at[ix[b]], semS[b]).wait()
        pltpu.make_async_copy(ones, cnt.at[ix[b]], semC[b]).wait()

    def _line(j):
        b = j % NBUF
        L_wait(j, b)
        S_start(j, b)
        if j - DS >= 0:
            S_wait(j - DS, (j - DS) % NBUF)
        if j + DL <= NLOAD - 1:
            L_start(j + DL, (j + DL) % NBUF)

    for j0 in range(DL):
        L_start(j0, j0 % NBUF)
    for k in range(NZ):
        pltpu.make_async_copy(zsrc, acc.at[pl.ds(s0 + k * SCAT, SCAT)],
                              semS[k % NBUF]).wait()
    pltpu.make_async_copy(zc, cnt.at[pl.ds(s0, SEG_PER_TILE)], semC[0]).wait()
    plsc.subcore_barrier()

    HEAD = NBUF
    TAIL0 = ((NLOAD - DL) // NBUF) * NBUF
    for j0 in range(HEAD):
        _line(j0)

    def _steady(i, _):
        for b in range(NBUF):
            j = NBUF * i + b
            L_wait(j, b)
            S_start(j, b)
            S_wait(j - DS, (b - DS) % NBUF)
            L_start(j + DL, (b + DL) % NBUF)
        return 0
    lax.fori_loop(HEAD // NBUF, TAIL0 // NBUF, _steady, 0)

    for j0 in range(TAIL0, NLOAD):
        _line(j0)
    for j0 in range(NLOAD - DS, NLOAD):
        S_wait(j0, j0 % NBUF)

    if REM:
        rr = base + NLOAD * SCAT
        pltpu.sync_copy(b_hbm.at[pl.ds(rr, REM)], ixr)
        pltpu.sync_copy(x_hbm.at[pl.ds(rr, REM)], xb0.at[pl.ds(0, REM)])
        pltpu.sync_copy(xb0.at[pl.ds(0, REM)], acc.at[ixr], add=True)
        pltpu.sync_copy(onesr, cnt.at[ixr], add=True)
    plsc.subcore_barrier()

    pltpu.sync_copy(acc.at[pl.ds(s0, SEG_PER_TILE)],
                    sums_out.at[cid, pl.ds(s0, SEG_PER_TILE)])
    pltpu.sync_copy(cnt.at[pl.ds(s0, SEG_PER_TILE)],
                    cnts_out.at[cid, pl.ds(s0, SEG_PER_TILE)])


@jax.jit
def _sc_aggregate(x, batch):
    mesh = plsc.VectorSubcoreMesh(core_axis_name="c", subcore_axis_name="s")
    f = pl.kernel(
        _sc_body,
        out_type=(jax.ShapeDtypeStruct((NC, S_PAD, D), jnp.float32),
                  jax.ShapeDtypeStruct((NC, S_PAD), jnp.float32)),
        mesh=mesh,
        scratch_types=[
            pltpu.VMEM_SHARED((S_PAD, D), jnp.float32),
            pltpu.VMEM_SHARED((S_PAD,), jnp.float32),
        ] + [pltpu.VMEM((SCAT, D), jnp.float32)] * NBUF
          + [pltpu.VMEM((SCAT,), jnp.int32)] * NBUF + [
            pltpu.VMEM((SCAT,), jnp.float32),
            pltpu.VMEM((SEG_PER_TILE,), jnp.float32),
            pltpu.VMEM((16,), jnp.int32),
            pltpu.VMEM((16,), jnp.float32),
        ] + [pltpu.SemaphoreType.DMA] * (3 * NBUF),
    )
    return f(x, batch)


BLK = 2000


def _fin_body(sums_ref, cnts_ref, y_ref, w_ref, b_ref, o_ref):
    s = sums_ref[0] + sums_ref[1]
    cnt = cnts_ref[:, 0:1] + cnts_ref[:, 1:2]
    ctx = jnp.dot(y_ref[...], w_ref[...].T,
                  preferred_element_type=jnp.float32) + b_ref[...]
    mean = s / jnp.maximum(cnt, 1.0)
    o_ref[...] = mean + jnp.where(cnt > 0.0, ctx, 0.0)


@jax.jit
def _finalize(sums, cnts_t, y2, W_c, b2):
    return pl.pallas_call(
        _fin_body,
        grid=(S // BLK,),
        in_specs=[
            pl.BlockSpec((NC, BLK, D), lambda i: (0, i, 0)),
            pl.BlockSpec((BLK, NC), lambda i: (i, 0)),
            pl.BlockSpec((1, D), lambda i: (0, 0)),
            pl.BlockSpec((D, D), lambda i: (0, 0)),
            pl.BlockSpec((1, D), lambda i: (0, 0)),
        ],
        out_specs=pl.BlockSpec((BLK, D), lambda i: (i, 0)),
        out_shape=jax.ShapeDtypeStruct((S, D), jnp.float32),
    )(sums, cnts_t, y2, W_c, b2)


def kernel(x, y, batch, W_c, b_c):
    batch32 = batch.astype(jnp.int32)
    sums, cnts = _sc_aggregate(x, batch32)
    return _finalize(sums, cnts.T, y.reshape(1, D), W_c, b_c.reshape(1, D))
